# baseline (speedup 1.0000x reference)
import functools

import jax
import jax.numpy as jnp
from jax import lax
from jax.experimental import pallas as pl
from jax.experimental.pallas import tpu as pltpu
from jax.experimental.pallas import tpu_sc as plsc

EMBED_DIM = 64
NUM_CLS = 16
SEQ = 200
CHUNKS = ((0, 104), (104, 96))
LANES = 16
NQ = EMBED_DIM // LANES


@functools.cache
def _sc_pool(batch):
  info = plsc.get_sparse_core_info()
  num_workers = info.num_cores * info.num_subcores
  bpw = batch // num_workers
  mesh = plsc.VectorSubcoreMesh(core_axis_name="c", subcore_axis_name="s")

  @functools.partial(
      pl.kernel,
      out_type=jax.ShapeDtypeStruct((batch, EMBED_DIM), jnp.float32),
      mesh=mesh,
      scratch_types=[
          pltpu.VMEM((bpw * SEQ,), jnp.int32),
          pltpu.VMEM((2, SEQ, EMBED_DIM), jnp.float32),
          pltpu.VMEM((bpw, EMBED_DIM), jnp.float32),
          pltpu.SemaphoreType.DMA,
      ],
      compiler_params=pltpu.CompilerParams(use_tc_tiling_on_sc=False),
  )
  def sc_pool(ids_hbm, emb_hbm, out_hbm, idx_v, rows_v, pooled_v, sem):
    wid = lax.axis_index("s") * info.num_cores + lax.axis_index("c")
    base = wid * bpw
    pltpu.sync_copy(ids_hbm.at[pl.ds(base * SEQ, bpw * SEQ)], idx_v)

    def row_dmas(row, buf):
      return [
          pltpu.make_async_copy(
              emb_hbm.at[idx_v.at[pl.ds(row * SEQ + off, sz)]],
              rows_v.at[buf, pl.ds(off, sz)],
              sem,
          )
          for off, sz in CHUNKS
      ]

    def fire(row, buf):
      for dma in row_dmas(row, buf):
        dma.start()

    def drain_reduce(row, buf):
      for dma in row_dmas(row, buf):
        dma.wait()
      zero = jnp.zeros((LANES,), jnp.float32)

      def body(r, acc):
        return tuple(
            acc[q] + rows_v[buf, r, pl.ds(q * LANES, LANES)]
            for q in range(NQ)
        )

      acc = lax.fori_loop(0, SEQ, body, (zero,) * NQ, unroll=2)
      for q in range(NQ):
        pooled_v[row, pl.ds(q * LANES, LANES)] = acc[q]

    fire(0, 0)

    def outer(g, carry):
      row = g * 2
      fire(row + 1, 1)
      drain_reduce(row, 0)

      @pl.when(row + 2 < bpw)
      def _():
        fire(row + 2, 0)

      drain_reduce(row + 1, 1)
      return carry

    lax.fori_loop(0, bpw // 2, outer, 0)
    pltpu.sync_copy(pooled_v, out_hbm.at[pl.ds(base, bpw)])

  return sc_pool


def _tc_head(pooled_sum, w_t, bias):
  def body(p_ref, w_ref, b_ref, o_ref):
    o_ref[...] = (
        jnp.dot(p_ref[...], w_ref[...], preferred_element_type=jnp.float32)
        * (1.0 / SEQ)
        + b_ref[...]
    )

  return pl.pallas_call(
      body,
      out_shape=jax.ShapeDtypeStruct(
          (pooled_sum.shape[0], NUM_CLS), jnp.float32
      ),
  )(pooled_sum, w_t, bias)


def kernel(text_ids, length, emb, W, b):
  del length
  pooled_sum = _sc_pool(text_ids.shape[0])(text_ids.reshape(-1), emb)
  return _tc_head(pooled_sum, W.T, b.reshape(1, NUM_CLS))


# speedup vs baseline: 1.0576x; 1.0576x over previous
import functools

import jax
import jax.numpy as jnp
from jax import lax
from jax.experimental import pallas as pl
from jax.experimental.pallas import tpu as pltpu
from jax.experimental.pallas import tpu_sc as plsc

EMBED_DIM = 64
NUM_CLS = 16
SEQ = 200
CHUNKS = ((0, 104), (104, 96))
LANES = 16
NQ = EMBED_DIM // LANES


@functools.cache
def _sc_pool(batch):
  info = plsc.get_sparse_core_info()
  num_workers = info.num_cores * info.num_subcores
  bpw = batch // num_workers
  mesh = plsc.VectorSubcoreMesh(core_axis_name="c", subcore_axis_name="s")

  nbuf = 4

  @functools.partial(
      pl.kernel,
      out_type=jax.ShapeDtypeStruct((batch, EMBED_DIM), jnp.float32),
      mesh=mesh,
      scratch_types=[
          pltpu.VMEM((bpw, SEQ), jnp.int32),
          pltpu.VMEM((nbuf, SEQ, EMBED_DIM), jnp.float32),
          pltpu.VMEM((bpw, EMBED_DIM), jnp.float32),
          pltpu.SemaphoreType.DMA,
      ],
      compiler_params=pltpu.CompilerParams(use_tc_tiling_on_sc=False),
  )
  def sc_pool(ids_hbm, emb_hbm, out_hbm, idx_v, rows_v, pooled_v, sem):
    wid = lax.axis_index("s") * info.num_cores + lax.axis_index("c")
    base = wid * bpw
    pltpu.sync_copy(ids_hbm.at[pl.ds(base, bpw)], idx_v)

    def row_dmas(row, buf):
      return [
          pltpu.make_async_copy(
              emb_hbm.at[idx_v.at[row, pl.ds(off, sz)]],
              rows_v.at[buf, pl.ds(off, sz)],
              sem,
          )
          for off, sz in CHUNKS
      ]

    def fire(row, buf):
      for dma in row_dmas(row, buf):
        dma.start()

    def drain_reduce(row, buf):
      for dma in row_dmas(row, buf):
        dma.wait()
      zero = jnp.zeros((LANES,), jnp.float32)

      def body(r, acc):
        return tuple(
            acc[q] + rows_v[buf, r, pl.ds(q * LANES, LANES)]
            for q in range(NQ)
        )

      acc = lax.fori_loop(0, SEQ, body, (zero,) * NQ, unroll=4)
      for q in range(NQ):
        pooled_v[row, pl.ds(q * LANES, LANES)] = acc[q] * (1.0 / SEQ)

    for i in range(nbuf - 1):
      fire(i, i)

    def outer(g, carry):
      for b in range(nbuf):
        row = g * nbuf + b

        @pl.when(row + nbuf - 1 < bpw)
        def _():
          fire(row + nbuf - 1, (b + nbuf - 1) % nbuf)

        drain_reduce(row, b)
      return carry

    lax.fori_loop(0, bpw // nbuf, outer, 0)
    pltpu.sync_copy(pooled_v, out_hbm.at[pl.ds(base, bpw)])

  return sc_pool


def _tc_head(pooled_sum, w_t, bias):
  def body(p_ref, w_ref, b_ref, o_ref):
    o_ref[...] = (
        jnp.dot(p_ref[...], w_ref[...], preferred_element_type=jnp.float32)
        + b_ref[...]
    )

  return pl.pallas_call(
      body,
      out_shape=jax.ShapeDtypeStruct(
          (pooled_sum.shape[0], NUM_CLS), jnp.float32
      ),
  )(pooled_sum, w_t, bias)


def kernel(text_ids, length, emb, W, b):
  del length
  pooled = _sc_pool(text_ids.shape[0])(text_ids, emb)
  return _tc_head(pooled, W.T, b.reshape(1, NUM_CLS))


# speedup vs baseline: 1.0632x; 1.0053x over previous
import functools

import jax
import jax.numpy as jnp
from jax import lax
from jax.experimental import pallas as pl
from jax.experimental.pallas import tpu as pltpu
from jax.experimental.pallas import tpu_sc as plsc

EMBED_DIM = 64
NUM_CLS = 16
SEQ = 200
CHUNKS = ((0, 200),)
LANES = 16
NQ = EMBED_DIM // LANES


@functools.cache
def _sc_pool(batch):
  info = plsc.get_sparse_core_info()
  num_workers = info.num_cores * info.num_subcores
  bpw = batch // num_workers
  mesh = plsc.VectorSubcoreMesh(core_axis_name="c", subcore_axis_name="s")

  nbuf = 4

  @functools.partial(
      pl.kernel,
      out_type=jax.ShapeDtypeStruct((batch, EMBED_DIM), jnp.float32),
      mesh=mesh,
      scratch_types=[
          pltpu.VMEM((bpw, SEQ), jnp.int32),
          pltpu.VMEM((nbuf, SEQ, EMBED_DIM), jnp.float32),
          pltpu.VMEM((bpw, EMBED_DIM), jnp.float32),
          pltpu.SemaphoreType.DMA,
      ],
      compiler_params=pltpu.CompilerParams(use_tc_tiling_on_sc=False),
  )
  def sc_pool(ids_hbm, emb_hbm, out_hbm, idx_v, rows_v, pooled_v, sem):
    wid = lax.axis_index("s") * info.num_cores + lax.axis_index("c")
    base = wid * bpw
    pltpu.sync_copy(ids_hbm.at[pl.ds(base, bpw)], idx_v)

    def row_dmas(row, buf):
      return [
          pltpu.make_async_copy(
              emb_hbm.at[idx_v.at[row, pl.ds(off, sz)]],
              rows_v.at[buf, pl.ds(off, sz)],
              sem,
          )
          for off, sz in CHUNKS
      ]

    def fire(row, buf):
      for dma in row_dmas(row, buf):
        dma.start()

    def drain_reduce(row, buf):
      for dma in row_dmas(row, buf):
        dma.wait()
      zero = jnp.zeros((LANES,), jnp.float32)

      def body(r, acc):
        return tuple(
            acc[q] + rows_v[buf, r, pl.ds(q * LANES, LANES)]
            for q in range(NQ)
        )

      acc = lax.fori_loop(0, SEQ, body, (zero,) * NQ, unroll=4)
      for q in range(NQ):
        pooled_v[row, pl.ds(q * LANES, LANES)] = acc[q] * (1.0 / SEQ)

    for i in range(nbuf - 1):
      fire(i, i)

    def outer(g, carry):
      for b in range(nbuf):
        row = g * nbuf + b

        @pl.when(row + nbuf - 1 < bpw)
        def _():
          fire(row + nbuf - 1, (b + nbuf - 1) % nbuf)

        drain_reduce(row, b)
      return carry

    lax.fori_loop(0, bpw // nbuf, outer, 0)
    pltpu.sync_copy(pooled_v, out_hbm.at[pl.ds(base, bpw)])

  return sc_pool


def _tc_head(pooled_sum, w_t, bias):
  def body(p_ref, w_ref, b_ref, o_ref):
    o_ref[...] = (
        jnp.dot(p_ref[...], w_ref[...], preferred_element_type=jnp.float32)
        + b_ref[...]
    )

  return pl.pallas_call(
      body,
      out_shape=jax.ShapeDtypeStruct(
          (pooled_sum.shape[0], NUM_CLS), jnp.float32
      ),
  )(pooled_sum, w_t, bias)


def kernel(text_ids, length, emb, W, b):
  del length
  pooled = _sc_pool(text_ids.shape[0])(text_ids, emb)
  return _tc_head(pooled, W.T, b.reshape(1, NUM_CLS))
